# baseline (speedup 1.0000x reference)
import functools

import jax
import jax.numpy as jnp
from jax import lax
from jax.experimental import pallas as pl
from jax.experimental.pallas import tpu as pltpu
from jax.experimental.pallas import tpu_sc as plsc

N = 10000
E = 320000
D = 128
NC = 2
NS = 16
NW = NC * NS
CK = 128
NCH = E // CK
CPW = NCH // NW
CREM = NCH % NW
CMAX = CPW + 1
NPAD = 10240
RPS = NPAD // NS
DEGW = 16
L = 16
NBUF = 3
RB2 = 2
IB = 4

_MESH = dict(core_axis_name="c", subcore_axis_name="s", num_cores=NC,
             num_subcores=NS)


def _zero_vmem_2d(ref, rows, cols):
    z = jnp.zeros((L,), jnp.float32)

    def body(k, _):
        i = k // (cols // L)
        j = k % (cols // L)
        ref[i, pl.ds(j * L, L)] = z
        return _

    lax.fori_loop(0, rows * (cols // L), body, None)


@functools.partial(
    pl.kernel,
    out_type=jax.ShapeDtypeStruct((NC, NPAD, DEGW), jnp.float32),
    mesh=plsc.VectorSubcoreMesh(**_MESH),
    scratch_types=[
        pltpu.VMEM((CMAX, CK), jnp.int32),
        pltpu.VMEM((CK, DEGW), jnp.float32),
        pltpu.VMEM_SHARED((NPAD, DEGW), jnp.float32),
        pltpu.SemaphoreType.DMA((NBUF,)),
        pltpu.SemaphoreType.DMA,
    ],
)
def _sc_degree(ei_hbm, out_hbm, idx_v, ones_v, acc_sh, ssem, psem):
    c = lax.axis_index("c")
    s = lax.axis_index("s")
    wid = c * NS + s
    ncw = jnp.where(wid < CREM, CPW + 1, CPW)

    def pre(t, _):
        base = E + (wid + t * NW) * CK
        pltpu.make_async_copy(ei_hbm.at[pl.ds(base, CK)],
                              idx_v.at[t], psem).start()
        return _

    def pre_wait(t, _):
        pltpu.make_async_copy(ei_hbm.at[pl.ds(E + wid * CK, CK)],
                              idx_v.at[0], psem).wait()
        return _

    lax.fori_loop(0, ncw, pre, None)

    _zero_vmem_2d(ones_v, CK, DEGW)

    def zinit(j, _):
        pltpu.sync_copy(ones_v, acc_sh.at[pl.ds(s * RPS + j * CK, CK)])
        return _

    lax.fori_loop(0, RPS // CK, zinit, None)

    one = jnp.full((L,), 1.0, jnp.float32)

    def fill(i, _):
        ones_v[i, :] = one
        return _

    lax.fori_loop(0, CK, fill, None)
    lax.fori_loop(0, ncw, pre_wait, None)
    plsc.subcore_barrier()

    def scat_wait(b):
        pltpu.make_async_copy(ones_v, acc_sh.at[idx_v.at[0]],
                              ssem.at[b]).wait()

    def step(t, _):
        b = lax.rem(t, NBUF)

        @pl.when(t >= NBUF)
        def _():
            scat_wait(b)

        pltpu.async_copy(ones_v, acc_sh.at[idx_v.at[t]], ssem.at[b],
                         add=True)
        return _

    lax.fori_loop(0, ncw, step, None)
    for t in range(NBUF):
        scat_wait(lax.rem(ncw - 1 - t, NBUF))
    plsc.subcore_barrier()

    def wb(j, _):
        r0 = s * RPS + j * CK
        pltpu.sync_copy(acc_sh.at[pl.ds(r0, CK)], ones_v)
        pltpu.sync_copy(ones_v, out_hbm.at[c, pl.ds(r0, CK)])
        return _

    lax.fori_loop(0, RPS // CK, wb, None)


_RB = 2000


def _prep_body(x_ref, w_ref, dp_ref, g_ref):
    h = jnp.dot(x_ref[...], w_ref[...], preferred_element_type=jnp.float32)
    deg = dp_ref[0, :, 0:1] + dp_ref[1, :, 0:1] + 1.0
    g_ref[...] = h * lax.rsqrt(deg)


def _tc_prep(x, W, degpart):
    return pl.pallas_call(
        _prep_body,
        grid=(N // _RB,),
        in_specs=[
            pl.BlockSpec((_RB, D), lambda i: (i, 0)),
            pl.BlockSpec((D, D), lambda i: (0, 0)),
            pl.BlockSpec((NC, _RB, DEGW), lambda i: (0, i, 0)),
        ],
        out_specs=pl.BlockSpec((_RB, D), lambda i: (i, 0)),
        out_shape=jax.ShapeDtypeStruct((N, D), jnp.float32),
    )(x, W, degpart)


@functools.partial(
    pl.kernel,
    out_type=jax.ShapeDtypeStruct((NC, NPAD, D), jnp.float32),
    mesh=plsc.VectorSubcoreMesh(**_MESH),
    scratch_types=[
        pltpu.VMEM((IB, CK), jnp.int32),
        pltpu.VMEM((IB, CK), jnp.int32),
        pltpu.VMEM((RB2, CK, D), jnp.float32),
        pltpu.VMEM_SHARED((NPAD, D), jnp.float32),
        pltpu.SemaphoreType.DMA((IB,)),
        pltpu.SemaphoreType.DMA((RB2,)),
        pltpu.SemaphoreType.DMA((RB2,)),
    ],
)
def _sc_edges(g_hbm, ei_hbm, out_hbm, sidx_v, didx_v, rows_v, acc_sh, isem,
              gsem, ssem):
    c = lax.axis_index("c")
    s = lax.axis_index("s")
    wid = c * NS + s
    ncw = jnp.where(wid < CREM, CPW + 1, CPW)

    def idx_start(t, ib):
        base = (wid + t * NW) * CK
        pltpu.make_async_copy(ei_hbm.at[pl.ds(base, CK)], sidx_v.at[ib],
                              isem.at[ib]).start()
        pltpu.make_async_copy(ei_hbm.at[pl.ds(E + base, CK)], didx_v.at[ib],
                              isem.at[ib]).start()

    def idx_wait(ib):
        pltpu.make_async_copy(ei_hbm.at[pl.ds(0, CK)], sidx_v.at[ib],
                              isem.at[ib]).wait()
        pltpu.make_async_copy(ei_hbm.at[pl.ds(0, CK)], didx_v.at[ib],
                              isem.at[ib]).wait()

    def gath(ib, b):
        return pltpu.make_async_copy(g_hbm.at[sidx_v.at[ib]], rows_v.at[b],
                                     gsem.at[b])

    def scat_wait(b):
        pltpu.make_async_copy(rows_v.at[b], acc_sh.at[didx_v.at[0]],
                              ssem.at[b]).wait()

    idx_start(0, 0)
    idx_start(1, 1)
    idx_start(2, 2)

    _zero_vmem_2d(rows_v.at[0], CK, D)

    def zinit(j, _):
        pltpu.sync_copy(rows_v.at[0], acc_sh.at[pl.ds(s * RPS + j * CK, CK)])
        return _

    lax.fori_loop(0, RPS // CK, zinit, None)
    plsc.subcore_barrier()

    idx_wait(0)
    gath(0, 0).start()

    def step(t, _):
        b = lax.rem(t, RB2)
        b1 = lax.rem(t + 1, RB2)

        @pl.when(t + 1 < ncw)
        def _():
            @pl.when(t + 1 >= RB2)
            def _():
                scat_wait(b1)

            @pl.when(t + 3 < ncw)
            def _():
                idx_start(t + 3, lax.rem(t + 3, IB))

            ib1 = lax.rem(t + 1, IB)
            idx_wait(ib1)
            gath(ib1, b1).start()

        gath(lax.rem(t, IB), b).wait()
        pltpu.async_copy(rows_v.at[b], acc_sh.at[didx_v.at[lax.rem(t, IB)]],
                         ssem.at[b], add=True)
        return _

    lax.fori_loop(0, ncw, step, None)
    for t in range(RB2):
        scat_wait(lax.rem(ncw - 1 - t, RB2))
    plsc.subcore_barrier()

    def wb(j, _):
        r0 = s * RPS + j * CK
        pltpu.sync_copy(acc_sh.at[pl.ds(r0, CK)], rows_v.at[0])
        pltpu.sync_copy(rows_v.at[0], out_hbm.at[c, pl.ds(r0, CK)])
        return _

    lax.fori_loop(0, RPS // CK, wb, None)


def _final_body(ap_ref, g_ref, dp_ref, b_ref, a_ref, o_ref):
    deg = dp_ref[0, :, 0:1] + dp_ref[1, :, 0:1] + 1.0
    dinv = lax.rsqrt(deg)
    z = dinv * (ap_ref[0] + ap_ref[1] + g_ref[...]) + b_ref[...]
    o_ref[...] = jnp.where(z >= 0, z, a_ref[0, 0] * z)


def _tc_final(accpart, g, degpart, b2, a2):
    return pl.pallas_call(
        _final_body,
        grid=(N // _RB,),
        in_specs=[
            pl.BlockSpec((NC, _RB, D), lambda i: (0, i, 0)),
            pl.BlockSpec((_RB, D), lambda i: (i, 0)),
            pl.BlockSpec((NC, _RB, DEGW), lambda i: (0, i, 0)),
            pl.BlockSpec((1, D), lambda i: (0, 0)),
            pl.BlockSpec((1, 1), lambda i: (0, 0)),
        ],
        out_specs=pl.BlockSpec((_RB, D), lambda i: (i, 0)),
        out_shape=jax.ShapeDtypeStruct((N, D), jnp.float32),
    )(accpart, g, degpart, b2, a2)


def kernel(x, edge_index, W, b, prelu_a):
    eiflat = edge_index.astype(jnp.int32).reshape(2 * E)
    degpart = _sc_degree(eiflat)
    g = _tc_prep(x, W, degpart)
    accpart = _sc_edges(g, eiflat)
    return _tc_final(accpart, g, degpart, b.reshape(1, D),
                     prelu_a.reshape(1, 1))
